# Initial kernel scaffold; baseline (speedup 1.0000x reference)
import jax
import jax.numpy as jnp
from jax.experimental import pallas as pl


def kernel(node_features, edge_index, target_nodes, picked_nodes, actions, w_n2l, bias_n2l, bias_picked, conv_W, conv_b, lin1_W, lin1_b, linout_W, linout_b):
    raise NotImplementedError("write your pallas kernel here")


# speedup vs baseline: 24.9179x; 24.9179x over previous
import functools

import jax
import jax.numpy as jnp
from jax import lax
from jax.experimental import pallas as pl
from jax.experimental.pallas import tpu as pltpu
from jax.experimental.pallas import tpu_sc as plsc

N = 10000
NPAD = 10240
E = 320000
D = 128
F = 64
B = 8
NC, NS, LANES = 2, 16, 16
NW = NC * NS
ECHUNK = 2000
NCHUNKS = E // ECHUNK
ITERS = ECHUNK // LANES

_mesh = plsc.VectorSubcoreMesh(core_axis_name="c", subcore_axis_name="s",
                               num_cores=NC, num_subcores=NS)
_sc_params = pltpu.CompilerParams(needs_layout_passes=False)


def _rsqrt_newton(x):
    i = plsc.bitcast(x, jnp.int32)
    i = 0x5F3759DF - (i >> 1)
    y = plsc.bitcast(i, jnp.float32)
    for _ in range(3):
        y = y * (1.5 - 0.5 * x * y * y)
    return y


@functools.partial(
    pl.kernel,
    out_type=jax.ShapeDtypeStruct((NC * NPAD,), jnp.int32),
    mesh=_mesh,
    compiler_params=_sc_params,
    scratch_types=[
        pltpu.VMEM((NPAD,), jnp.int32),
        pltpu.VMEM((ECHUNK,), jnp.int32),
        pltpu.VMEM((ECHUNK,), jnp.int32),
        pltpu.VMEM((640,), jnp.int32),
        pltpu.VMEM((640,), jnp.int32),
        pltpu.VMEM_SHARED((NS, NPAD), jnp.int32),
    ])
def _deg_kernel(esrc, edst, out, cnt, sbuf, dbuf, sacc, stmp, shared):
    cid = lax.axis_index("c")
    sid = lax.axis_index("s")
    wid = sid * NC + cid
    zeros16 = jnp.zeros((LANES,), jnp.int32)

    def z(i, _):
        cnt[pl.ds(i * LANES, LANES)] = zeros16
        return 0
    lax.fori_loop(0, NPAD // LANES, z, 0)

    ones16 = jnp.ones((LANES,), jnp.int32)
    base = wid * (E // NW)

    def chunk(ci, _):
        off = base + ci * ECHUNK
        pltpu.sync_copy(esrc.at[pl.ds(off, ECHUNK)], sbuf)
        pltpu.sync_copy(edst.at[pl.ds(off, ECHUNK)], dbuf)

        def it(i, _):
            sl = pl.ds(i * LANES, LANES)
            plsc.addupdate_scatter(cnt, [sbuf[sl]], ones16)
            plsc.addupdate_scatter(cnt, [dbuf[sl]], ones16)
            return 0
        lax.fori_loop(0, ITERS, it, 0)
        return 0
    lax.fori_loop(0, (E // NW) // ECHUNK, chunk, 0)

    pltpu.sync_copy(cnt, shared.at[sid])
    plsc.subcore_barrier()
    soff = sid * 640
    pltpu.sync_copy(shared.at[0, pl.ds(soff, 640)], sacc)

    def red(k, _):
        pltpu.sync_copy(shared.at[k, pl.ds(soff, 640)], stmp)

        def add16(i, _):
            sl = pl.ds(i * LANES, LANES)
            sacc[sl] = sacc[sl] + stmp[sl]
            return 0
        lax.fori_loop(0, 640 // LANES, add16, 0)
        return 0
    lax.fori_loop(1, NS, red, 0)
    pltpu.sync_copy(sacc, out.at[pl.ds(cid * NPAD + soff, 640)])


@functools.partial(
    pl.kernel,
    out_type=(jax.ShapeDtypeStruct((F * N,), jnp.float32),
              jax.ShapeDtypeStruct((B * NPAD,), jnp.float32),
              jax.ShapeDtypeStruct((NPAD,), jnp.float32)),
    mesh=_mesh,
    compiler_params=_sc_params,
    scratch_types=[
        pltpu.VMEM((2 * N,), jnp.float32),
        pltpu.VMEM((2 * N,), jnp.float32),
        pltpu.VMEM((NPAD,), jnp.float32),
        pltpu.VMEM((NPAD,), jnp.float32),
        pltpu.VMEM((NPAD,), jnp.int32),
        pltpu.VMEM((NPAD,), jnp.int32),
        pltpu.VMEM((ECHUNK,), jnp.int32),
        pltpu.VMEM((ECHUNK,), jnp.int32),
        pltpu.VMEM((LANES,), jnp.int32),
    ])
def _spmm1_kernel(esrc, edst, x0t, degp, picked, p1t, ap, dinv_out,
                  xbuf, acc, dinv, apbuf, deg0, deg1, sbuf, dbuf, pvbuf):
    cid = lax.axis_index("c")
    sid = lax.axis_index("s")
    wid = sid * NC + cid

    pltpu.sync_copy(degp.at[pl.ds(0, NPAD)], deg0)
    pltpu.sync_copy(degp.at[pl.ds(NPAD, NPAD)], deg1)

    def dv(i, _):
        sl = pl.ds(i * LANES, LANES)
        k = deg0[sl] + deg1[sl] + 1
        dinv[sl] = _rsqrt_newton(k.astype(jnp.float32))
        return 0
    lax.fori_loop(0, NPAD // LANES, dv, 0)

    pltpu.sync_copy(x0t.at[pl.ds(2 * wid * N, N)], xbuf.at[pl.ds(0, N)])
    pltpu.sync_copy(x0t.at[pl.ds((2 * wid + 1) * N, N)], xbuf.at[pl.ds(N, N)])
    pltpu.sync_copy(picked.at[pl.ds(jnp.minimum(wid, B - 1) * LANES, LANES)],
                    pvbuf)

    zf = jnp.zeros((LANES,), jnp.float32)

    def z(i, _):
        acc[pl.ds(i * LANES, LANES)] = zf
        return 0
    lax.fori_loop(0, 2 * N // LANES, z, 0)

    def z2(i, _):
        apbuf[pl.ds(i * LANES, LANES)] = zf
        return 0
    lax.fori_loop(0, NPAD // LANES, z2, 0)

    def edge_pass(with_ap):
        def chunk(ci, _):
            off = ci * ECHUNK
            pltpu.sync_copy(esrc.at[pl.ds(off, ECHUNK)], sbuf)
            pltpu.sync_copy(edst.at[pl.ds(off, ECHUNK)], dbuf)

            def it(i, _):
                sl = pl.ds(i * LANES, LANES)
                s = sbuf[sl]
                d2 = dbuf[sl]
                ws = plsc.load_gather(dinv, [s])
                wd = plsc.load_gather(dinv, [d2])
                ew = ws * wd
                for f in range(2):
                    so = s if f == 0 else s + f * N
                    do = d2 if f == 0 else d2 + f * N
                    xs = plsc.load_gather(xbuf, [so])
                    plsc.addupdate_scatter(acc, [do], ew * xs)
                    xd = plsc.load_gather(xbuf, [do])
                    plsc.addupdate_scatter(acc, [so], ew * xd)
                if with_ap:
                    pv = pvbuf[...]
                    plsc.addupdate_scatter(apbuf, [d2], ew, mask=s == pv)
                    plsc.addupdate_scatter(apbuf, [s], ew, mask=d2 == pv)
                return 0
            lax.fori_loop(0, ITERS, it, 0)
            return 0
        lax.fori_loop(0, NCHUNKS, chunk, 0)

    @pl.when(wid < B)
    def _():
        edge_pass(True)

    @pl.when(wid >= B)
    def _():
        edge_pass(False)

    def selfl(i, _):
        sl = pl.ds(i * LANES, LANES)
        di = dinv[sl]
        w = di * di
        for f in range(2):
            s2 = pl.ds(f * N + i * LANES, LANES)
            acc[s2] = acc[s2] + w * xbuf[s2]
        return 0
    lax.fori_loop(0, N // LANES, selfl, 0)

    ii0 = lax.iota(jnp.int32, LANES)

    @pl.when(wid < B)
    def _():
        pv = pvbuf[...]
        dp = plsc.load_gather(dinv, [pv])
        plsc.addupdate_scatter(apbuf, [pv], dp * dp, mask=ii0 == 0)
        pltpu.sync_copy(apbuf, ap.at[pl.ds(wid * NPAD, NPAD)])

    pltpu.sync_copy(acc.at[pl.ds(0, N)], p1t.at[pl.ds(2 * wid * N, N)])
    pltpu.sync_copy(acc.at[pl.ds(N, N)], p1t.at[pl.ds((2 * wid + 1) * N, N)])

    @pl.when(wid == B)
    def _():
        pltpu.sync_copy(dinv, dinv_out)


@functools.partial(
    pl.kernel,
    out_type=jax.ShapeDtypeStruct((B * F * N,), jnp.float32),
    mesh=_mesh,
    compiler_params=_sc_params,
    scratch_types=[
        pltpu.VMEM((4 * N,), jnp.float32),
        pltpu.VMEM((4 * N,), jnp.float32),
        pltpu.VMEM((NPAD,), jnp.float32),
        pltpu.VMEM((ECHUNK,), jnp.int32),
        pltpu.VMEM((ECHUNK,), jnp.int32),
    ])
def _spmm2_kernel(esrc, edst, x1t, dinv_in, p2t, xb, acc, dinv, sbuf, dbuf):
    cid = lax.axis_index("c")
    sid = lax.axis_index("s")
    wid = sid * NC + cid

    pltpu.sync_copy(dinv_in, dinv)
    zf = jnp.zeros((LANES,), jnp.float32)

    def pass_body(g, _):
        bb = (g % 2) * 4
        col = (g // 2) * NW + wid
        for k in range(4):
            pltpu.sync_copy(x1t.at[pl.ds(((bb + k) * F + col) * N, N)],
                            xb.at[pl.ds(k * N, N)])

        def z(i, _):
            acc[pl.ds(i * LANES, LANES)] = zf
            return 0
        lax.fori_loop(0, 4 * N // LANES, z, 0)

        def chunk(ci, _):
            off = ci * ECHUNK
            pltpu.sync_copy(esrc.at[pl.ds(off, ECHUNK)], sbuf)
            pltpu.sync_copy(edst.at[pl.ds(off, ECHUNK)], dbuf)

            def it(i, _):
                sl = pl.ds(i * LANES, LANES)
                s = sbuf[sl]
                d2 = dbuf[sl]
                ws = plsc.load_gather(dinv, [s])
                wd = plsc.load_gather(dinv, [d2])
                ew = ws * wd
                for k in range(4):
                    so = s if k == 0 else s + k * N
                    do = d2 if k == 0 else d2 + k * N
                    xs = plsc.load_gather(xb, [so])
                    plsc.addupdate_scatter(acc, [do], ew * xs)
                    xd = plsc.load_gather(xb, [do])
                    plsc.addupdate_scatter(acc, [so], ew * xd)
                return 0
            lax.fori_loop(0, ITERS, it, 0)
            return 0
        lax.fori_loop(0, NCHUNKS, chunk, 0)

        def selfl(i, _):
            sl = pl.ds(i * LANES, LANES)
            di = dinv[sl]
            w = di * di
            for k in range(4):
                s2 = pl.ds(k * N + i * LANES, LANES)
                acc[s2] = acc[s2] + w * xb[s2]
            return 0
        lax.fori_loop(0, N // LANES, selfl, 0)

        for k in range(4):
            pltpu.sync_copy(acc.at[pl.ds(k * N, N)],
                            p2t.at[pl.ds(((bb + k) * F + col) * N, N)])
        return 0
    lax.fori_loop(0, 4, pass_body, 0)


def _n2l_body(nf_ref, w_ref, b_ref, xint_ref, x0t_ref):
    xt = lax.dot_general(w_ref[...], nf_ref[...], (((0,), (1,)), ((), ())),
                         preferred_element_type=jnp.float32)
    xt = xt + b_ref[...]
    xint_ref[...] = xt
    x0t_ref[...] = jnp.maximum(xt, 0.0)


def _lvl1_body(p1_ref, xint_ref, ap_ref, ut_ref, oh_ref, w_ref, cb_ref,
               cc_ref, out_ref):
    m = lax.dot_general(w_ref[...], p1_ref[...], (((0,), (0,)), ((), ())),
                        preferred_element_type=jnp.float32)
    m = m + cb_ref[...] + xint_ref[...]
    m = m + ut_ref[0] * ap_ref[0]
    m = m + cc_ref[...] * oh_ref[0]
    out_ref[...] = jnp.maximum(m, 0.0)[None]


def _readout_body(p2_ref, xint_ref, oh_ref, rt_ref, w_ref, cb_ref, cc_ref,
                  l1w_ref, l1b_ref, low_ref, lob_ref, out_ref):
    m = lax.dot_general(w_ref[...], p2_ref[0], (((0,), (0,)), ((), ())),
                        preferred_element_type=jnp.float32)
    x2 = jnp.maximum(m + cb_ref[...] + xint_ref[...]
                     + cc_ref[...] * oh_ref[0][0], 0.0)
    y = lax.dot_general(rt_ref[0], x2, (((1,), (1,)), ((), ())),
                        preferred_element_type=jnp.float32)
    te = y[0:1]
    ae = y[1:2]
    ge = y[2:3]
    esa = jnp.concatenate([ae, ge], axis=1)
    h = jnp.maximum(jnp.dot(esa, l1w_ref[...],
                            preferred_element_type=jnp.float32) + l1b_ref[...],
                    0.0)
    raw = jnp.dot(h, low_ref[...],
                  preferred_element_type=jnp.float32) + lob_ref[...]
    out_ref[...] = jnp.sum(raw * te).reshape(1, 1, 1)


def kernel(node_features, edge_index, target_nodes, picked_nodes, actions,
           w_n2l, bias_n2l, bias_picked, conv_W, conv_b,
           lin1_W, lin1_b, linout_W, linout_b):
    ei = edge_index.astype(jnp.int32)
    esrc = ei[0].reshape(E)
    edst = ei[1].reshape(E)
    c = bias_picked[0]
    cb_col = conv_b.reshape(F, 1)
    cc_col = c.reshape(F, 1)

    xint, x0t = pl.pallas_call(
        _n2l_body,
        out_shape=(jax.ShapeDtypeStruct((F, N), jnp.float32),
                   jax.ShapeDtypeStruct((F, N), jnp.float32)),
    )(node_features, w_n2l, bias_n2l.reshape(F, 1))

    degp = _deg_kernel(esrc, edst)

    picked_b = jnp.broadcast_to(
        picked_nodes.astype(jnp.int32)[:, None], (B, LANES)).reshape(-1)
    p1t, ap, dinv = _spmm1_kernel(esrc, edst, x0t.reshape(-1), degp, picked_b)
    p1t = p1t.reshape(F, N)
    ap = ap.reshape(B, NPAD)[:, :N].reshape(B, 1, N)

    xp = jnp.take(xint, picked_nodes, axis=1)
    d0 = jax.nn.relu(xp + c[:, None]) - jax.nn.relu(xp)
    ut = (conv_W.T @ d0).T.reshape(B, F, 1)
    iota_n = jnp.arange(N, dtype=jnp.int32)
    onehot = (picked_nodes[:, None] == iota_n[None, :]).astype(
        jnp.float32).reshape(B, 1, N)

    x1t = pl.pallas_call(
        _lvl1_body,
        grid=(B,),
        in_specs=[
            pl.BlockSpec((F, N), lambda b: (0, 0)),
            pl.BlockSpec((F, N), lambda b: (0, 0)),
            pl.BlockSpec((1, 1, N), lambda b: (b, 0, 0)),
            pl.BlockSpec((1, F, 1), lambda b: (b, 0, 0)),
            pl.BlockSpec((1, 1, N), lambda b: (b, 0, 0)),
            pl.BlockSpec((F, F), lambda b: (0, 0)),
            pl.BlockSpec((F, 1), lambda b: (0, 0)),
            pl.BlockSpec((F, 1), lambda b: (0, 0)),
        ],
        out_specs=pl.BlockSpec((1, F, N), lambda b: (b, 0, 0)),
        out_shape=jax.ShapeDtypeStruct((B, F, N), jnp.float32),
    )(p1t, xint, ap, ut, onehot, conv_W, cb_col, cc_col)

    p2t = _spmm2_kernel(esrc, edst, x1t.reshape(-1), dinv).reshape(B, F, N)

    oh_t = (target_nodes[:, None] == iota_n[None, :]).astype(jnp.float32)
    oh_a = (actions[:, None] == iota_n[None, :]).astype(jnp.float32)
    mean_row = jnp.full((B, N), 1.0 / N, dtype=jnp.float32)
    rt = jnp.stack([oh_t, oh_a, mean_row, jnp.zeros((B, N), jnp.float32)],
                   axis=1)

    out = pl.pallas_call(
        _readout_body,
        grid=(B,),
        in_specs=[
            pl.BlockSpec((1, F, N), lambda b: (b, 0, 0)),
            pl.BlockSpec((F, N), lambda b: (0, 0)),
            pl.BlockSpec((1, 1, N), lambda b: (b, 0, 0)),
            pl.BlockSpec((1, 4, N), lambda b: (b, 0, 0)),
            pl.BlockSpec((F, F), lambda b: (0, 0)),
            pl.BlockSpec((F, 1), lambda b: (0, 0)),
            pl.BlockSpec((F, 1), lambda b: (0, 0)),
            pl.BlockSpec((D, F), lambda b: (0, 0)),
            pl.BlockSpec((1, F), lambda b: (0, 0)),
            pl.BlockSpec((F, F), lambda b: (0, 0)),
            pl.BlockSpec((1, F), lambda b: (0, 0)),
        ],
        out_specs=pl.BlockSpec((1, 1, 1), lambda b: (b, 0, 0)),
        out_shape=jax.ShapeDtypeStruct((B, 1, 1), jnp.float32),
    )(p2t, xint, onehot, rt, conv_W, cb_col, cc_col,
      lin1_W, lin1_b.reshape(1, F), linout_W, linout_b.reshape(1, F))

    return out.reshape(B)


# speedup vs baseline: 29.4471x; 1.1818x over previous
import functools

import jax
import jax.numpy as jnp
from jax import lax
from jax.experimental import pallas as pl
from jax.experimental.pallas import tpu as pltpu
from jax.experimental.pallas import tpu_sc as plsc

N = 10000
NPAD = 10240
E = 320000
D = 128
F = 64
B = 8
NC, NS, LANES = 2, 16, 16
NW = NC * NS
ECHUNK = 8000
NCHUNKS = E // ECHUNK
ITERS = ECHUNK // LANES
KCH = 2000
KITERS = KCH // LANES

_mesh = plsc.VectorSubcoreMesh(core_axis_name="c", subcore_axis_name="s",
                               num_cores=NC, num_subcores=NS)
_sc_params = pltpu.CompilerParams(needs_layout_passes=False)


def _rsqrt_newton(x):
    i = plsc.bitcast(x, jnp.int32)
    i = 0x5F3759DF - (i >> 1)
    y = plsc.bitcast(i, jnp.float32)
    for _ in range(3):
        y = y * (1.5 - 0.5 * x * y * y)
    return y


@functools.partial(
    pl.kernel,
    out_type=jax.ShapeDtypeStruct((NC * NPAD,), jnp.int32),
    mesh=_mesh,
    compiler_params=_sc_params,
    scratch_types=[
        pltpu.VMEM((NPAD,), jnp.int32),
        pltpu.VMEM((KCH,), jnp.int32),
        pltpu.VMEM((KCH,), jnp.int32),
        pltpu.VMEM((640,), jnp.int32),
        pltpu.VMEM((640,), jnp.int32),
        pltpu.VMEM_SHARED((NS, NPAD), jnp.int32),
    ])
def _deg_kernel(esrc, edst, out, cnt, sbuf, dbuf, sacc, stmp, shared):
    cid = lax.axis_index("c")
    sid = lax.axis_index("s")
    wid = sid * NC + cid
    zeros16 = jnp.zeros((LANES,), jnp.int32)

    def z(i, _):
        cnt[pl.ds(i * LANES, LANES)] = zeros16
        return 0
    lax.fori_loop(0, NPAD // LANES, z, 0)

    ones16 = jnp.ones((LANES,), jnp.int32)
    base = wid * (E // NW)

    def chunk(ci, _):
        off = base + ci * KCH
        pltpu.sync_copy(esrc.at[pl.ds(off, KCH)], sbuf)
        pltpu.sync_copy(edst.at[pl.ds(off, KCH)], dbuf)

        def it(i, _):
            sl = pl.ds(i * LANES, LANES)
            plsc.addupdate_scatter(cnt, [sbuf[sl]], ones16)
            plsc.addupdate_scatter(cnt, [dbuf[sl]], ones16)
            return 0
        lax.fori_loop(0, KITERS, it, 0)
        return 0
    lax.fori_loop(0, (E // NW) // KCH, chunk, 0)

    pltpu.sync_copy(cnt, shared.at[sid])
    plsc.subcore_barrier()
    soff = sid * 640
    pltpu.sync_copy(shared.at[0, pl.ds(soff, 640)], sacc)

    def red(k, _):
        pltpu.sync_copy(shared.at[k, pl.ds(soff, 640)], stmp)

        def add16(i, _):
            sl = pl.ds(i * LANES, LANES)
            sacc[sl] = sacc[sl] + stmp[sl]
            return 0
        lax.fori_loop(0, 640 // LANES, add16, 0)
        return 0
    lax.fori_loop(1, NS, red, 0)
    pltpu.sync_copy(sacc, out.at[pl.ds(cid * NPAD + soff, 640)])


def _edge_stream(esrc, edst, sb0, db0, sb1, db1, sem0, sem1, it_body):
    def start(ci, sb, db, sem):
        pltpu.async_copy(esrc.at[pl.ds(ci * ECHUNK, ECHUNK)], sb, sem)
        pltpu.async_copy(edst.at[pl.ds(ci * ECHUNK, ECHUNK)], db, sem)

    def wait(ci, sb, db, sem):
        pltpu.make_async_copy(
            esrc.at[pl.ds(ci * ECHUNK, ECHUNK)], sb, sem).wait()
        pltpu.make_async_copy(
            edst.at[pl.ds(ci * ECHUNK, ECHUNK)], db, sem).wait()

    def proc(sb, db):
        def it(i, _):
            sl = pl.ds(i * LANES, LANES)
            it_body(sb[sl], db[sl])
            return 0
        lax.fori_loop(0, ITERS, it, 0)

    start(0, sb0, db0, sem0)

    def body2(j, _):
        c0 = 2 * j
        wait(c0, sb0, db0, sem0)
        start(c0 + 1, sb1, db1, sem1)
        proc(sb0, db0)
        wait(c0 + 1, sb1, db1, sem1)

        @pl.when(c0 + 2 < NCHUNKS)
        def _():
            start(c0 + 2, sb0, db0, sem0)
        proc(sb1, db1)
        return 0
    lax.fori_loop(0, NCHUNKS // 2, body2, 0)


@functools.partial(
    pl.kernel,
    out_type=(jax.ShapeDtypeStruct((F * N,), jnp.float32),
              jax.ShapeDtypeStruct((B * NPAD,), jnp.float32),
              jax.ShapeDtypeStruct((NPAD,), jnp.float32)),
    mesh=_mesh,
    compiler_params=_sc_params,
    scratch_types=[
        pltpu.VMEM((2 * N,), jnp.float32),
        pltpu.VMEM((2 * N,), jnp.float32),
        pltpu.VMEM((NPAD,), jnp.float32),
        pltpu.VMEM((NPAD,), jnp.float32),
        pltpu.VMEM((NPAD,), jnp.int32),
        pltpu.VMEM((NPAD,), jnp.int32),
        pltpu.VMEM((ECHUNK,), jnp.int32),
        pltpu.VMEM((ECHUNK,), jnp.int32),
        pltpu.VMEM((ECHUNK,), jnp.int32),
        pltpu.VMEM((ECHUNK,), jnp.int32),
        pltpu.VMEM((LANES,), jnp.int32),
        pltpu.SemaphoreType.DMA,
        pltpu.SemaphoreType.DMA,
    ])
def _spmm1_kernel(esrc, edst, x0t, degp, picked, p1t, ap, dinv_out,
                  xbuf, acc, dinv, apbuf, deg0, deg1, sb0, db0, sb1, db1,
                  pvbuf, sem0, sem1):
    cid = lax.axis_index("c")
    sid = lax.axis_index("s")
    wid = sid * NC + cid

    pltpu.sync_copy(degp.at[pl.ds(0, NPAD)], deg0)
    pltpu.sync_copy(degp.at[pl.ds(NPAD, NPAD)], deg1)

    def dv(i, _):
        sl = pl.ds(i * LANES, LANES)
        k = deg0[sl] + deg1[sl] + 1
        dinv[sl] = _rsqrt_newton(k.astype(jnp.float32))
        return 0
    lax.fori_loop(0, NPAD // LANES, dv, 0)

    pltpu.sync_copy(x0t.at[pl.ds(2 * wid * N, N)], xbuf.at[pl.ds(0, N)])
    pltpu.sync_copy(x0t.at[pl.ds((2 * wid + 1) * N, N)], xbuf.at[pl.ds(N, N)])
    pltpu.sync_copy(picked.at[pl.ds(jnp.minimum(wid, B - 1) * LANES, LANES)],
                    pvbuf)

    zf = jnp.zeros((LANES,), jnp.float32)

    def z(i, _):
        acc[pl.ds(i * LANES, LANES)] = zf
        return 0
    lax.fori_loop(0, 2 * N // LANES, z, 0)

    def z2(i, _):
        apbuf[pl.ds(i * LANES, LANES)] = zf
        return 0
    lax.fori_loop(0, NPAD // LANES, z2, 0)

    def edge_pass(with_ap):
        def it_body(s, d2):
            ws = plsc.load_gather(dinv, [s])
            wd = plsc.load_gather(dinv, [d2])
            ew = ws * wd
            for f in range(2):
                so = s if f == 0 else s + f * N
                do = d2 if f == 0 else d2 + f * N
                xs = plsc.load_gather(xbuf, [so])
                plsc.addupdate_scatter(acc, [do], ew * xs)
                xd = plsc.load_gather(xbuf, [do])
                plsc.addupdate_scatter(acc, [so], ew * xd)
            if with_ap:
                pv = pvbuf[...]
                plsc.addupdate_scatter(apbuf, [d2], ew, mask=s == pv)
                plsc.addupdate_scatter(apbuf, [s], ew, mask=d2 == pv)
        _edge_stream(esrc, edst, sb0, db0, sb1, db1, sem0, sem1, it_body)

    @pl.when(wid < B)
    def _():
        edge_pass(True)

    @pl.when(wid >= B)
    def _():
        edge_pass(False)

    def selfl(i, _):
        sl = pl.ds(i * LANES, LANES)
        di = dinv[sl]
        w = di * di
        for f in range(2):
            s2 = pl.ds(f * N + i * LANES, LANES)
            acc[s2] = acc[s2] + w * xbuf[s2]
        return 0
    lax.fori_loop(0, N // LANES, selfl, 0)

    ii0 = lax.iota(jnp.int32, LANES)

    @pl.when(wid < B)
    def _():
        pv = pvbuf[...]
        dp = plsc.load_gather(dinv, [pv])
        plsc.addupdate_scatter(apbuf, [pv], dp * dp, mask=ii0 == 0)
        pltpu.sync_copy(apbuf, ap.at[pl.ds(wid * NPAD, NPAD)])

    pltpu.sync_copy(acc.at[pl.ds(0, N)], p1t.at[pl.ds(2 * wid * N, N)])
    pltpu.sync_copy(acc.at[pl.ds(N, N)], p1t.at[pl.ds((2 * wid + 1) * N, N)])

    @pl.when(wid == B)
    def _():
        pltpu.sync_copy(dinv, dinv_out)


@functools.partial(
    pl.kernel,
    out_type=jax.ShapeDtypeStruct((B * F * N,), jnp.float32),
    mesh=_mesh,
    compiler_params=_sc_params,
    scratch_types=[
        pltpu.VMEM((4 * N,), jnp.float32),
        pltpu.VMEM((4 * N,), jnp.float32),
        pltpu.VMEM((NPAD,), jnp.float32),
        pltpu.VMEM((ECHUNK,), jnp.int32),
        pltpu.VMEM((ECHUNK,), jnp.int32),
        pltpu.VMEM((ECHUNK,), jnp.int32),
        pltpu.VMEM((ECHUNK,), jnp.int32),
        pltpu.SemaphoreType.DMA,
        pltpu.SemaphoreType.DMA,
    ])
def _spmm2_kernel(esrc, edst, x1t, dinv_in, p2t, xb, acc, dinv,
                  sb0, db0, sb1, db1, sem0, sem1):
    cid = lax.axis_index("c")
    sid = lax.axis_index("s")
    wid = sid * NC + cid

    pltpu.sync_copy(dinv_in, dinv)
    zf = jnp.zeros((LANES,), jnp.float32)

    def pass_body(g, _):
        bb = (g % 2) * 4
        col = (g // 2) * NW + wid
        for k in range(4):
            pltpu.sync_copy(x1t.at[pl.ds(((bb + k) * F + col) * N, N)],
                            xb.at[pl.ds(k * N, N)])

        def z(i, _):
            acc[pl.ds(i * LANES, LANES)] = zf
            return 0
        lax.fori_loop(0, 4 * N // LANES, z, 0)

        def it_body(s, d2):
            ws = plsc.load_gather(dinv, [s])
            wd = plsc.load_gather(dinv, [d2])
            ew = ws * wd
            for k in range(4):
                so = s if k == 0 else s + k * N
                do = d2 if k == 0 else d2 + k * N
                xs = plsc.load_gather(xb, [so])
                plsc.addupdate_scatter(acc, [do], ew * xs)
                xd = plsc.load_gather(xb, [do])
                plsc.addupdate_scatter(acc, [so], ew * xd)
        _edge_stream(esrc, edst, sb0, db0, sb1, db1, sem0, sem1, it_body)

        def selfl(i, _):
            sl = pl.ds(i * LANES, LANES)
            di = dinv[sl]
            w = di * di
            for k in range(4):
                s2 = pl.ds(k * N + i * LANES, LANES)
                acc[s2] = acc[s2] + w * xb[s2]
            return 0
        lax.fori_loop(0, N // LANES, selfl, 0)

        for k in range(4):
            pltpu.sync_copy(acc.at[pl.ds(k * N, N)],
                            p2t.at[pl.ds(((bb + k) * F + col) * N, N)])
        return 0
    lax.fori_loop(0, 4, pass_body, 0)


def _n2l_body(nf_ref, w_ref, b_ref, xint_ref, x0t_ref):
    xt = lax.dot_general(w_ref[...], nf_ref[...], (((0,), (1,)), ((), ())),
                         preferred_element_type=jnp.float32)
    xt = xt + b_ref[...]
    xint_ref[...] = xt
    x0t_ref[...] = jnp.maximum(xt, 0.0)


def _lvl1_body(p1_ref, xint_ref, ap_ref, ut_ref, oh_ref, w_ref, cb_ref,
               cc_ref, out_ref):
    m = lax.dot_general(w_ref[...], p1_ref[...], (((0,), (0,)), ((), ())),
                        preferred_element_type=jnp.float32)
    m = m + cb_ref[...] + xint_ref[...]
    m = m + ut_ref[0] * ap_ref[0]
    m = m + cc_ref[...] * oh_ref[0]
    out_ref[...] = jnp.maximum(m, 0.0)[None]


def _readout_body(p2_ref, xint_ref, oh_ref, rt_ref, w_ref, cb_ref, cc_ref,
                  l1w_ref, l1b_ref, low_ref, lob_ref, out_ref):
    m = lax.dot_general(w_ref[...], p2_ref[0], (((0,), (0,)), ((), ())),
                        preferred_element_type=jnp.float32)
    x2 = jnp.maximum(m + cb_ref[...] + xint_ref[...]
                     + cc_ref[...] * oh_ref[0][0], 0.0)
    y = lax.dot_general(rt_ref[0], x2, (((1,), (1,)), ((), ())),
                        preferred_element_type=jnp.float32)
    te = y[0:1]
    ae = y[1:2]
    ge = y[2:3]
    esa = jnp.concatenate([ae, ge], axis=1)
    h = jnp.maximum(jnp.dot(esa, l1w_ref[...],
                            preferred_element_type=jnp.float32) + l1b_ref[...],
                    0.0)
    raw = jnp.dot(h, low_ref[...],
                  preferred_element_type=jnp.float32) + lob_ref[...]
    out_ref[...] = jnp.sum(raw * te).reshape(1, 1, 1)


def kernel(node_features, edge_index, target_nodes, picked_nodes, actions,
           w_n2l, bias_n2l, bias_picked, conv_W, conv_b,
           lin1_W, lin1_b, linout_W, linout_b):
    ei = edge_index.astype(jnp.int32)
    esrc = ei[0].reshape(E)
    edst = ei[1].reshape(E)
    c = bias_picked[0]
    cb_col = conv_b.reshape(F, 1)
    cc_col = c.reshape(F, 1)

    xint, x0t = pl.pallas_call(
        _n2l_body,
        out_shape=(jax.ShapeDtypeStruct((F, N), jnp.float32),
                   jax.ShapeDtypeStruct((F, N), jnp.float32)),
    )(node_features, w_n2l, bias_n2l.reshape(F, 1))

    degp = _deg_kernel(esrc, edst)

    picked_b = jnp.broadcast_to(
        picked_nodes.astype(jnp.int32)[:, None], (B, LANES)).reshape(-1)
    p1t, ap, dinv = _spmm1_kernel(esrc, edst, x0t.reshape(-1), degp, picked_b)
    p1t = p1t.reshape(F, N)
    ap = ap.reshape(B, NPAD)[:, :N].reshape(B, 1, N)

    xp = jnp.take(xint, picked_nodes, axis=1)
    d0 = jax.nn.relu(xp + c[:, None]) - jax.nn.relu(xp)
    ut = (conv_W.T @ d0).T.reshape(B, F, 1)
    iota_n = jnp.arange(N, dtype=jnp.int32)
    onehot = (picked_nodes[:, None] == iota_n[None, :]).astype(
        jnp.float32).reshape(B, 1, N)

    x1t = pl.pallas_call(
        _lvl1_body,
        grid=(B,),
        in_specs=[
            pl.BlockSpec((F, N), lambda b: (0, 0)),
            pl.BlockSpec((F, N), lambda b: (0, 0)),
            pl.BlockSpec((1, 1, N), lambda b: (b, 0, 0)),
            pl.BlockSpec((1, F, 1), lambda b: (b, 0, 0)),
            pl.BlockSpec((1, 1, N), lambda b: (b, 0, 0)),
            pl.BlockSpec((F, F), lambda b: (0, 0)),
            pl.BlockSpec((F, 1), lambda b: (0, 0)),
            pl.BlockSpec((F, 1), lambda b: (0, 0)),
        ],
        out_specs=pl.BlockSpec((1, F, N), lambda b: (b, 0, 0)),
        out_shape=jax.ShapeDtypeStruct((B, F, N), jnp.float32),
    )(p1t, xint, ap, ut, onehot, conv_W, cb_col, cc_col)

    p2t = _spmm2_kernel(esrc, edst, x1t.reshape(-1), dinv).reshape(B, F, N)

    oh_t = (target_nodes[:, None] == iota_n[None, :]).astype(jnp.float32)
    oh_a = (actions[:, None] == iota_n[None, :]).astype(jnp.float32)
    mean_row = jnp.full((B, N), 1.0 / N, dtype=jnp.float32)
    rt = jnp.stack([oh_t, oh_a, mean_row, jnp.zeros((B, N), jnp.float32)],
                   axis=1)

    out = pl.pallas_call(
        _readout_body,
        grid=(B,),
        in_specs=[
            pl.BlockSpec((1, F, N), lambda b: (b, 0, 0)),
            pl.BlockSpec((F, N), lambda b: (0, 0)),
            pl.BlockSpec((1, 1, N), lambda b: (b, 0, 0)),
            pl.BlockSpec((1, 4, N), lambda b: (b, 0, 0)),
            pl.BlockSpec((F, F), lambda b: (0, 0)),
            pl.BlockSpec((F, 1), lambda b: (0, 0)),
            pl.BlockSpec((F, 1), lambda b: (0, 0)),
            pl.BlockSpec((D, F), lambda b: (0, 0)),
            pl.BlockSpec((1, F), lambda b: (0, 0)),
            pl.BlockSpec((F, F), lambda b: (0, 0)),
            pl.BlockSpec((1, F), lambda b: (0, 0)),
        ],
        out_specs=pl.BlockSpec((1, 1, 1), lambda b: (b, 0, 0)),
        out_shape=jax.ShapeDtypeStruct((B, 1, 1), jnp.float32),
    )(p2t, xint, onehot, rt, conv_W, cb_col, cc_col,
      lin1_W, lin1_b.reshape(1, F), linout_W, linout_b.reshape(1, F))

    return out.reshape(B)


# speedup vs baseline: 83.7767x; 2.8450x over previous
import functools

import jax
import jax.numpy as jnp
from jax import lax
from jax.experimental import pallas as pl
from jax.experimental.pallas import tpu as pltpu
from jax.experimental.pallas import tpu_sc as plsc

N = 10000
NPAD = 10240
E = 320000
D = 128
F = 64
B = 8
NC, NS, LANES = 2, 16, 16
NW = NC * NS
ECHUNK = 8000
NCHUNKS = E // ECHUNK
ITERS = ECHUNK // LANES
KCH = 2000
KITERS = KCH // LANES

_mesh = plsc.VectorSubcoreMesh(core_axis_name="c", subcore_axis_name="s",
                               num_cores=NC, num_subcores=NS)
_sc_params = pltpu.CompilerParams(needs_layout_passes=False)


def _rsqrt_newton(x):
    i = plsc.bitcast(x, jnp.int32)
    i = 0x5F3759DF - (i >> 1)
    y = plsc.bitcast(i, jnp.float32)
    for _ in range(3):
        y = y * (1.5 - 0.5 * x * y * y)
    return y


@functools.partial(
    pl.kernel,
    out_type=jax.ShapeDtypeStruct((NC * NPAD,), jnp.int32),
    mesh=_mesh,
    compiler_params=_sc_params,
    scratch_types=[
        pltpu.VMEM((NPAD,), jnp.int32),
        pltpu.VMEM((KCH,), jnp.int32),
        pltpu.VMEM((KCH,), jnp.int32),
        pltpu.VMEM((640,), jnp.int32),
        pltpu.VMEM((640,), jnp.int32),
        pltpu.VMEM_SHARED((NS, NPAD), jnp.int32),
    ])
def _deg_kernel(esrc, edst, out, cnt, sbuf, dbuf, sacc, stmp, shared):
    cid = lax.axis_index("c")
    sid = lax.axis_index("s")
    wid = sid * NC + cid
    zeros16 = jnp.zeros((LANES,), jnp.int32)

    def z(i, _):
        cnt[pl.ds(i * LANES, LANES)] = zeros16
        return 0
    lax.fori_loop(0, NPAD // LANES, z, 0)

    ones16 = jnp.ones((LANES,), jnp.int32)
    base = wid * (E // NW)

    def chunk(ci, _):
        off = base + ci * KCH
        pltpu.sync_copy(esrc.at[pl.ds(off, KCH)], sbuf)
        pltpu.sync_copy(edst.at[pl.ds(off, KCH)], dbuf)

        def it(i, _):
            sl = pl.ds(i * LANES, LANES)
            plsc.addupdate_scatter(cnt, [sbuf[sl]], ones16)
            plsc.addupdate_scatter(cnt, [dbuf[sl]], ones16)
            return 0
        lax.fori_loop(0, KITERS, it, 0)
        return 0
    lax.fori_loop(0, (E // NW) // KCH, chunk, 0)

    pltpu.sync_copy(cnt, shared.at[sid])
    plsc.subcore_barrier()
    soff = sid * 640
    pltpu.sync_copy(shared.at[0, pl.ds(soff, 640)], sacc)

    def red(k, _):
        pltpu.sync_copy(shared.at[k, pl.ds(soff, 640)], stmp)

        def add16(i, _):
            sl = pl.ds(i * LANES, LANES)
            sacc[sl] = sacc[sl] + stmp[sl]
            return 0
        lax.fori_loop(0, 640 // LANES, add16, 0)
        return 0
    lax.fori_loop(1, NS, red, 0)
    pltpu.sync_copy(sacc, out.at[pl.ds(cid * NPAD + soff, 640)])


def _edge_stream(esrc, edst, sb0, db0, sb1, db1, sem0, sem1, it_body):
    def start(ci, sb, db, sem):
        pltpu.async_copy(esrc.at[pl.ds(ci * ECHUNK, ECHUNK)], sb, sem)
        pltpu.async_copy(edst.at[pl.ds(ci * ECHUNK, ECHUNK)], db, sem)

    def wait(ci, sb, db, sem):
        pltpu.make_async_copy(
            esrc.at[pl.ds(ci * ECHUNK, ECHUNK)], sb, sem).wait()
        pltpu.make_async_copy(
            edst.at[pl.ds(ci * ECHUNK, ECHUNK)], db, sem).wait()

    def proc(sb, db):
        @plsc.parallel_loop(0, ECHUNK, LANES, unroll=4)
        def _(i):
            sl = pl.ds(i, LANES)
            it_body(sb[sl], db[sl])

    start(0, sb0, db0, sem0)

    def body2(j, _):
        c0 = 2 * j
        wait(c0, sb0, db0, sem0)
        start(c0 + 1, sb1, db1, sem1)
        proc(sb0, db0)
        wait(c0 + 1, sb1, db1, sem1)

        @pl.when(c0 + 2 < NCHUNKS)
        def _():
            start(c0 + 2, sb0, db0, sem0)
        proc(sb1, db1)
        return 0
    lax.fori_loop(0, NCHUNKS // 2, body2, 0)


@functools.partial(
    pl.kernel,
    out_type=(jax.ShapeDtypeStruct((F * N,), jnp.float32),
              jax.ShapeDtypeStruct((B * NPAD,), jnp.float32),
              jax.ShapeDtypeStruct((NPAD,), jnp.float32)),
    mesh=_mesh,
    compiler_params=_sc_params,
    scratch_types=[
        pltpu.VMEM((2 * N,), jnp.float32),
        pltpu.VMEM((2 * N,), jnp.float32),
        pltpu.VMEM((NPAD,), jnp.float32),
        pltpu.VMEM((NPAD,), jnp.float32),
        pltpu.VMEM((NPAD,), jnp.int32),
        pltpu.VMEM((NPAD,), jnp.int32),
        pltpu.VMEM((ECHUNK,), jnp.int32),
        pltpu.VMEM((ECHUNK,), jnp.int32),
        pltpu.VMEM((ECHUNK,), jnp.int32),
        pltpu.VMEM((ECHUNK,), jnp.int32),
        pltpu.VMEM((LANES,), jnp.int32),
        pltpu.SemaphoreType.DMA,
        pltpu.SemaphoreType.DMA,
    ])
def _spmm1_kernel(esrc, edst, x0t, degp, picked, p1t, ap, dinv_out,
                  xbuf, acc, dinv, apbuf, deg0, deg1, sb0, db0, sb1, db1,
                  pvbuf, sem0, sem1):
    cid = lax.axis_index("c")
    sid = lax.axis_index("s")
    wid = sid * NC + cid

    pltpu.sync_copy(degp.at[pl.ds(0, NPAD)], deg0)
    pltpu.sync_copy(degp.at[pl.ds(NPAD, NPAD)], deg1)

    def dv(i, _):
        sl = pl.ds(i * LANES, LANES)
        k = deg0[sl] + deg1[sl] + 1
        dinv[sl] = _rsqrt_newton(k.astype(jnp.float32))
        return 0
    lax.fori_loop(0, NPAD // LANES, dv, 0)

    pltpu.sync_copy(x0t.at[pl.ds(2 * wid * N, N)], xbuf.at[pl.ds(0, N)])
    pltpu.sync_copy(x0t.at[pl.ds((2 * wid + 1) * N, N)], xbuf.at[pl.ds(N, N)])
    pltpu.sync_copy(picked.at[pl.ds(jnp.minimum(wid, B - 1) * LANES, LANES)],
                    pvbuf)

    zf = jnp.zeros((LANES,), jnp.float32)

    def z(i, _):
        acc[pl.ds(i * LANES, LANES)] = zf
        return 0
    lax.fori_loop(0, 2 * N // LANES, z, 0)

    def z2(i, _):
        apbuf[pl.ds(i * LANES, LANES)] = zf
        return 0
    lax.fori_loop(0, NPAD // LANES, z2, 0)

    def edge_pass(with_ap):
        def it_body(s, d2):
            ws = plsc.load_gather(dinv, [s])
            wd = plsc.load_gather(dinv, [d2])
            ew = ws * wd
            for f in range(2):
                so = s if f == 0 else s + f * N
                do = d2 if f == 0 else d2 + f * N
                xs = plsc.load_gather(xbuf, [so])
                plsc.addupdate_scatter(acc, [do], ew * xs)
                xd = plsc.load_gather(xbuf, [do])
                plsc.addupdate_scatter(acc, [so], ew * xd)
            if with_ap:
                pv = pvbuf[...]
                plsc.addupdate_scatter(apbuf, [d2], ew, mask=s == pv)
                plsc.addupdate_scatter(apbuf, [s], ew, mask=d2 == pv)
        _edge_stream(esrc, edst, sb0, db0, sb1, db1, sem0, sem1, it_body)

    @pl.when(wid < B)
    def _():
        edge_pass(True)

    @pl.when(wid >= B)
    def _():
        edge_pass(False)

    def selfl(i, _):
        sl = pl.ds(i * LANES, LANES)
        di = dinv[sl]
        w = di * di
        for f in range(2):
            s2 = pl.ds(f * N + i * LANES, LANES)
            acc[s2] = acc[s2] + w * xbuf[s2]
        return 0
    lax.fori_loop(0, N // LANES, selfl, 0)

    ii0 = lax.iota(jnp.int32, LANES)

    @pl.when(wid < B)
    def _():
        pv = pvbuf[...]
        dp = plsc.load_gather(dinv, [pv])
        plsc.addupdate_scatter(apbuf, [pv], dp * dp, mask=ii0 == 0)
        pltpu.sync_copy(apbuf, ap.at[pl.ds(wid * NPAD, NPAD)])

    pltpu.sync_copy(acc.at[pl.ds(0, N)], p1t.at[pl.ds(2 * wid * N, N)])
    pltpu.sync_copy(acc.at[pl.ds(N, N)], p1t.at[pl.ds((2 * wid + 1) * N, N)])

    @pl.when(wid == B)
    def _():
        pltpu.sync_copy(dinv, dinv_out)


@functools.partial(
    pl.kernel,
    out_type=jax.ShapeDtypeStruct((B * F * N,), jnp.float32),
    mesh=_mesh,
    compiler_params=_sc_params,
    scratch_types=[
        pltpu.VMEM((4 * N,), jnp.float32),
        pltpu.VMEM((4 * N,), jnp.float32),
        pltpu.VMEM((NPAD,), jnp.float32),
        pltpu.VMEM((ECHUNK,), jnp.int32),
        pltpu.VMEM((ECHUNK,), jnp.int32),
        pltpu.VMEM((ECHUNK,), jnp.int32),
        pltpu.VMEM((ECHUNK,), jnp.int32),
        pltpu.SemaphoreType.DMA,
        pltpu.SemaphoreType.DMA,
    ])
def _spmm2_kernel(esrc, edst, x1t, dinv_in, p2t, xb, acc, dinv,
                  sb0, db0, sb1, db1, sem0, sem1):
    cid = lax.axis_index("c")
    sid = lax.axis_index("s")
    wid = sid * NC + cid

    pltpu.sync_copy(dinv_in, dinv)
    zf = jnp.zeros((LANES,), jnp.float32)

    def pass_body(g, _):
        bb = (g % 2) * 4
        col = (g // 2) * NW + wid
        for k in range(4):
            pltpu.sync_copy(x1t.at[pl.ds(((bb + k) * F + col) * N, N)],
                            xb.at[pl.ds(k * N, N)])

        def z(i, _):
            acc[pl.ds(i * LANES, LANES)] = zf
            return 0
        lax.fori_loop(0, 4 * N // LANES, z, 0)

        def it_body(s, d2):
            ws = plsc.load_gather(dinv, [s])
            wd = plsc.load_gather(dinv, [d2])
            ew = ws * wd
            for k in range(4):
                so = s if k == 0 else s + k * N
                do = d2 if k == 0 else d2 + k * N
                xs = plsc.load_gather(xb, [so])
                plsc.addupdate_scatter(acc, [do], ew * xs)
                xd = plsc.load_gather(xb, [do])
                plsc.addupdate_scatter(acc, [so], ew * xd)
        _edge_stream(esrc, edst, sb0, db0, sb1, db1, sem0, sem1, it_body)

        def selfl(i, _):
            sl = pl.ds(i * LANES, LANES)
            di = dinv[sl]
            w = di * di
            for k in range(4):
                s2 = pl.ds(k * N + i * LANES, LANES)
                acc[s2] = acc[s2] + w * xb[s2]
            return 0
        lax.fori_loop(0, N // LANES, selfl, 0)

        for k in range(4):
            pltpu.sync_copy(acc.at[pl.ds(k * N, N)],
                            p2t.at[pl.ds(((bb + k) * F + col) * N, N)])
        return 0
    lax.fori_loop(0, 4, pass_body, 0)


def _n2l_body(nf_ref, w_ref, b_ref, xint_ref, x0t_ref):
    xt = lax.dot_general(w_ref[...], nf_ref[...], (((0,), (1,)), ((), ())),
                         preferred_element_type=jnp.float32)
    xt = xt + b_ref[...]
    xint_ref[...] = xt
    x0t_ref[...] = jnp.maximum(xt, 0.0)


def _lvl1_body(p1_ref, xint_ref, ap_ref, ut_ref, oh_ref, w_ref, cb_ref,
               cc_ref, out_ref):
    m = lax.dot_general(w_ref[...], p1_ref[...], (((0,), (0,)), ((), ())),
                        preferred_element_type=jnp.float32)
    m = m + cb_ref[...] + xint_ref[...]
    m = m + ut_ref[0] * ap_ref[0]
    m = m + cc_ref[...] * oh_ref[0]
    out_ref[...] = jnp.maximum(m, 0.0)[None]


def _readout_body(p2_ref, xint_ref, oh_ref, rt_ref, w_ref, cb_ref, cc_ref,
                  l1w_ref, l1b_ref, low_ref, lob_ref, out_ref):
    m = lax.dot_general(w_ref[...], p2_ref[0], (((0,), (0,)), ((), ())),
                        preferred_element_type=jnp.float32)
    x2 = jnp.maximum(m + cb_ref[...] + xint_ref[...]
                     + cc_ref[...] * oh_ref[0][0], 0.0)
    y = lax.dot_general(rt_ref[0], x2, (((1,), (1,)), ((), ())),
                        preferred_element_type=jnp.float32)
    te = y[0:1]
    ae = y[1:2]
    ge = y[2:3]
    esa = jnp.concatenate([ae, ge], axis=1)
    h = jnp.maximum(jnp.dot(esa, l1w_ref[...],
                            preferred_element_type=jnp.float32) + l1b_ref[...],
                    0.0)
    raw = jnp.dot(h, low_ref[...],
                  preferred_element_type=jnp.float32) + lob_ref[...]
    out_ref[...] = jnp.sum(raw * te).reshape(1, 1, 1)


def kernel(node_features, edge_index, target_nodes, picked_nodes, actions,
           w_n2l, bias_n2l, bias_picked, conv_W, conv_b,
           lin1_W, lin1_b, linout_W, linout_b):
    ei = edge_index.astype(jnp.int32)
    esrc = ei[0].reshape(E)
    edst = ei[1].reshape(E)
    c = bias_picked[0]
    cb_col = conv_b.reshape(F, 1)
    cc_col = c.reshape(F, 1)

    xint, x0t = pl.pallas_call(
        _n2l_body,
        out_shape=(jax.ShapeDtypeStruct((F, N), jnp.float32),
                   jax.ShapeDtypeStruct((F, N), jnp.float32)),
    )(node_features, w_n2l, bias_n2l.reshape(F, 1))

    degp = _deg_kernel(esrc, edst)

    picked_b = jnp.broadcast_to(
        picked_nodes.astype(jnp.int32)[:, None], (B, LANES)).reshape(-1)
    p1t, ap, dinv = _spmm1_kernel(esrc, edst, x0t.reshape(-1), degp, picked_b)
    p1t = p1t.reshape(F, N)
    ap = ap.reshape(B, NPAD)[:, :N].reshape(B, 1, N)

    xp = jnp.take(xint, picked_nodes, axis=1)
    d0 = jax.nn.relu(xp + c[:, None]) - jax.nn.relu(xp)
    ut = (conv_W.T @ d0).T.reshape(B, F, 1)
    iota_n = jnp.arange(N, dtype=jnp.int32)
    onehot = (picked_nodes[:, None] == iota_n[None, :]).astype(
        jnp.float32).reshape(B, 1, N)

    x1t = pl.pallas_call(
        _lvl1_body,
        grid=(B,),
        in_specs=[
            pl.BlockSpec((F, N), lambda b: (0, 0)),
            pl.BlockSpec((F, N), lambda b: (0, 0)),
            pl.BlockSpec((1, 1, N), lambda b: (b, 0, 0)),
            pl.BlockSpec((1, F, 1), lambda b: (b, 0, 0)),
            pl.BlockSpec((1, 1, N), lambda b: (b, 0, 0)),
            pl.BlockSpec((F, F), lambda b: (0, 0)),
            pl.BlockSpec((F, 1), lambda b: (0, 0)),
            pl.BlockSpec((F, 1), lambda b: (0, 0)),
        ],
        out_specs=pl.BlockSpec((1, F, N), lambda b: (b, 0, 0)),
        out_shape=jax.ShapeDtypeStruct((B, F, N), jnp.float32),
    )(p1t, xint, ap, ut, onehot, conv_W, cb_col, cc_col)

    p2t = _spmm2_kernel(esrc, edst, x1t.reshape(-1), dinv).reshape(B, F, N)

    oh_t = (target_nodes[:, None] == iota_n[None, :]).astype(jnp.float32)
    oh_a = (actions[:, None] == iota_n[None, :]).astype(jnp.float32)
    mean_row = jnp.full((B, N), 1.0 / N, dtype=jnp.float32)
    rt = jnp.stack([oh_t, oh_a, mean_row, jnp.zeros((B, N), jnp.float32)],
                   axis=1)

    out = pl.pallas_call(
        _readout_body,
        grid=(B,),
        in_specs=[
            pl.BlockSpec((1, F, N), lambda b: (b, 0, 0)),
            pl.BlockSpec((F, N), lambda b: (0, 0)),
            pl.BlockSpec((1, 1, N), lambda b: (b, 0, 0)),
            pl.BlockSpec((1, 4, N), lambda b: (b, 0, 0)),
            pl.BlockSpec((F, F), lambda b: (0, 0)),
            pl.BlockSpec((F, 1), lambda b: (0, 0)),
            pl.BlockSpec((F, 1), lambda b: (0, 0)),
            pl.BlockSpec((D, F), lambda b: (0, 0)),
            pl.BlockSpec((1, F), lambda b: (0, 0)),
            pl.BlockSpec((F, F), lambda b: (0, 0)),
            pl.BlockSpec((1, F), lambda b: (0, 0)),
        ],
        out_specs=pl.BlockSpec((1, 1, 1), lambda b: (b, 0, 0)),
        out_shape=jax.ShapeDtypeStruct((B, 1, 1), jnp.float32),
    )(p2t, xint, onehot, rt, conv_W, cb_col, cc_col,
      lin1_W, lin1_b.reshape(1, F), linout_W, linout_b.reshape(1, F))

    return out.reshape(B)
